# baseline (speedup 1.0000x reference)
import functools

import jax
import jax.numpy as jnp
from jax import lax
from jax.experimental import pallas as pl
from jax.experimental.pallas import tpu as pltpu
from jax.experimental.pallas import tpu_sc as plsc

VOCAB = 1000000
HIDDEN = 64
OUT = 128
B = 4096
L = 50
LPAD = 128

NC = 2
NS = 16
NW = NC * NS
BPW = B // NW
CB = 2
SI = 64
CBL = CB * SI
NCH = BPW // CB
NBUF = 4


def _pool_sc(x_hbm, tbl_hbm, cnt_hbm, h_hbm, idx_v, kidx_v, rows,
             acc_v, cnt_s, sems):
    wid = lax.axis_index("s") * NC + lax.axis_index("c")
    base = wid * BPW
    pltpu.sync_copy(x_hbm.at[pl.ds(base, BPW)], idx_v)
    pltpu.sync_copy(cnt_hbm.at[pl.ds(base, BPW)], cnt_s.at[pl.ds(0, BPW)])

    def compact(j, carry):
        for q in range(4):
            kidx_v[pl.ds(j * SI + 16 * q, 16)] = idx_v[j, pl.ds(16 * q, 16)]
        return carry

    lax.fori_loop(0, BPW, compact, 0)

    def fire(c, b):
        pltpu.async_copy(tbl_hbm.at[kidx_v.at[pl.ds(c * CBL, CBL)]],
                         rows[b], sems[b])

    def wait(c, b):
        pltpu.make_async_copy(tbl_hbm.at[kidx_v.at[pl.ds(c * CBL, CBL)]],
                              rows[b], sems[b]).wait()

    def process(c, b):
        rbuf = rows[b]
        for i2 in range(CB):
            j = c * CB + i2
            n0 = cnt_s[pl.ds(j, 16)][0]
            roff = i2 * SI

            def red_even(r, accs):
                a0, a1, a2, a3 = accs
                a0 = a0 + rbuf[roff + r, pl.ds(0, 16)]
                a1 = a1 + rbuf[roff + r, pl.ds(16, 16)]
                a2 = a2 + rbuf[roff + r, pl.ds(32, 16)]
                a3 = a3 + rbuf[roff + r, pl.ds(48, 16)]
                return (a0, a1, a2, a3)

            def red_odd(r, accs):
                a0, a1, a2, a3 = accs
                a0 = a0 + rbuf[roff + r, pl.ds(64, 16)]
                a1 = a1 + rbuf[roff + r, pl.ds(80, 16)]
                a2 = a2 + rbuf[roff + r, pl.ds(96, 16)]
                a3 = a3 + rbuf[roff + r, pl.ds(112, 16)]
                return (a0, a1, a2, a3)

            z = jnp.zeros((16,), jnp.float32)
            a = lax.fori_loop(0, n0, red_even, (z,) * 4)
            a = lax.fori_loop(n0, L, red_odd, a)
            acc_v[j, pl.ds(0, 16)] = a[0]
            acc_v[j, pl.ds(16, 16)] = a[1]
            acc_v[j, pl.ds(32, 16)] = a[2]
            acc_v[j, pl.ds(48, 16)] = a[3]
            zv = jnp.zeros((16,), jnp.float32)
            acc_v[j, pl.ds(64, 16)] = zv
            acc_v[j, pl.ds(80, 16)] = zv
            acc_v[j, pl.ds(96, 16)] = zv
            acc_v[j, pl.ds(112, 16)] = zv

    for b in range(NBUF):
        fire(b, b)

    def group(i, carry):
        g = i * NBUF
        for b in range(NBUF):
            c = g + b
            wait(c, b)
            process(c, b)
            fire(c + NBUF, b)
        return carry

    lax.fori_loop(0, (NCH - NBUF) // NBUF, group, 0)

    for b in range(NBUF):
        c = NCH - NBUF + b
        wait(c, b)
        process(c, b)

    pltpu.sync_copy(acc_v, h_hbm.at[pl.ds(base, BPW)])


def _pool_body(x_hbm, tbl_hbm, cnt_hbm, h_hbm, idx_v, kidx_v,
               r0, r1, r2, r3, acc_v, cnt_s, s0, s1, s2, s3):
    _pool_sc(x_hbm, tbl_hbm, cnt_hbm, h_hbm, idx_v, kidx_v,
             (r0, r1, r2, r3), acc_v, cnt_s, (s0, s1, s2, s3))


@jax.jit
def _pool(kidx_pad, tbl_view, counts):
    mesh = plsc.VectorSubcoreMesh(core_axis_name="c", subcore_axis_name="s")
    return pl.kernel(
        _pool_body,
        mesh=mesh,
        compiler_params=pltpu.CompilerParams(needs_layout_passes=False),
        out_type=jax.ShapeDtypeStruct((B, LPAD), jnp.float32),
        scratch_types=(
            [
                pltpu.VMEM((BPW, LPAD), jnp.int32),
                pltpu.VMEM((BPW * SI,), jnp.int32),
            ]
            + [pltpu.VMEM((CBL, 128), jnp.float32) for _ in range(NBUF)]
            + [pltpu.VMEM((BPW, LPAD), jnp.float32)]
            + [pltpu.VMEM((BPW + 16,), jnp.int32)]
            + [pltpu.SemaphoreType.DMA for _ in range(NBUF)]
        ),
    )(kidx_pad, tbl_view, counts)


def _mm_body(h_ref, w_ref, b_ref, o_ref):
    o_ref[...] = (
        lax.dot_general(
            h_ref[...], w_ref[...],
            dimension_numbers=(((1,), (1,)), ((), ())),
            preferred_element_type=jnp.float32,
        )
        + b_ref[...]
    )


@jax.jit
def _linear(h, W2, b2d):
    bm = 512
    return pl.pallas_call(
        _mm_body,
        out_shape=jax.ShapeDtypeStruct((B, OUT), jnp.float32),
        grid=(B // bm,),
        in_specs=[
            pl.BlockSpec((bm, LPAD), lambda i: (i, 0)),
            pl.BlockSpec((OUT, LPAD), lambda i: (0, 0)),
            pl.BlockSpec((1, OUT), lambda i: (0, 0)),
        ],
        out_specs=pl.BlockSpec((bm, OUT), lambda i: (i, 0)),
    )(h, W2, b2d)


def kernel(x, emb_table, W, b):
    xi = x.astype(jnp.int32)
    lane = jnp.arange(LPAD, dtype=jnp.int32)[None, :]
    x_pad = jnp.pad(xi, ((0, 0), (0, LPAD - L)))
    real = lane < L
    odd = (x_pad & 1).astype(jnp.int32) * jnp.where(real, 1, 0)
    ev = jnp.where(real, 1 - odd, 0)
    n0 = jnp.sum(ev, axis=1).astype(jnp.int32)
    pos = jnp.where(
        real,
        jnp.where(ev > 0,
                  jnp.cumsum(ev, axis=1) - 1,
                  n0[:, None] + jnp.cumsum(odd, axis=1) - 1),
        lane,
    )
    rows_ix = jnp.broadcast_to(
        jnp.arange(B, dtype=jnp.int32)[:, None], (B, LPAD))
    kidx_pad = jnp.zeros((B, LPAD), jnp.int32).at[rows_ix, pos].set(
        x_pad >> 1, mode="drop")
    tbl_view = jnp.reshape(emb_table, (VOCAB // 2, 2 * HIDDEN))
    h = _pool(kidx_pad, tbl_view, n0)
    W2 = jnp.pad(W, ((0, 0), (0, LPAD - HIDDEN)))
    return _linear(h, W2, b.reshape(1, OUT))


# speedup vs baseline: 5.6387x; 5.6387x over previous
import functools

import jax
import jax.numpy as jnp
from jax import lax
from jax.experimental import pallas as pl
from jax.experimental.pallas import tpu as pltpu
from jax.experimental.pallas import tpu_sc as plsc

VOCAB = 1000000
HIDDEN = 64
OUT = 128
B = 4096
L = 50
LPAD = 52

NC = 2
NS = 16
NW = NC * NS
BPW = B // NW
CB = 2
CBL = CB * LPAD
NCH = BPW // CB
NBUF = 4


def _pool_sc(x_hbm, tbl_hbm, h_hbm, idx_v, rows, acc_v, sems):
    wid = lax.axis_index("s") * NC + lax.axis_index("c")
    base = wid * BPW
    pltpu.sync_copy(x_hbm.at[pl.ds(base * LPAD, BPW * LPAD)], idx_v)

    def fire(c, b):
        pltpu.async_copy(tbl_hbm.at[idx_v.at[pl.ds(c * CBL, CBL)]],
                         rows[b], sems[b])

    def wait(c, b):
        pltpu.make_async_copy(tbl_hbm.at[idx_v.at[pl.ds(c * CBL, CBL)]],
                              rows[b], sems[b]).wait()

    def process(c, b):
        rbuf = rows[b]
        for i2 in range(CB):
            roff = i2 * LPAD

            def red(r, accs):
                a0, a1, a2, a3, a4, a5, a6, a7 = accs
                r2 = roff + 2 * r
                a0 = a0 + rbuf[r2, pl.ds(0, 16)]
                a1 = a1 + rbuf[r2, pl.ds(16, 16)]
                a2 = a2 + rbuf[r2, pl.ds(32, 16)]
                a3 = a3 + rbuf[r2, pl.ds(48, 16)]
                a4 = a4 + rbuf[r2 + 1, pl.ds(0, 16)]
                a5 = a5 + rbuf[r2 + 1, pl.ds(16, 16)]
                a6 = a6 + rbuf[r2 + 1, pl.ds(32, 16)]
                a7 = a7 + rbuf[r2 + 1, pl.ds(48, 16)]
                return (a0, a1, a2, a3, a4, a5, a6, a7)

            z = jnp.zeros((16,), jnp.float32)
            a = lax.fori_loop(0, L // 2, red, (z,) * 8)
            j = c * CB + i2
            acc_v[j, pl.ds(0, 16)] = a[0] + a[4]
            acc_v[j, pl.ds(16, 16)] = a[1] + a[5]
            acc_v[j, pl.ds(32, 16)] = a[2] + a[6]
            acc_v[j, pl.ds(48, 16)] = a[3] + a[7]

    for b in range(NBUF):
        fire(b, b)

    def group(i, carry):
        g = i * NBUF
        for b in range(NBUF):
            c = g + b
            wait(c, b)
            process(c, b)
            fire(c + NBUF, b)
        return carry

    lax.fori_loop(0, (NCH - NBUF) // NBUF, group, 0)

    for b in range(NBUF):
        c = NCH - NBUF + b
        wait(c, b)
        process(c, b)

    pltpu.sync_copy(acc_v, h_hbm.at[pl.ds(base, BPW)])


def _pool_body(x_hbm, tbl_hbm, h_hbm, idx_v, r0, r1, r2, r3,
               acc_v, s0, s1, s2, s3):
    _pool_sc(x_hbm, tbl_hbm, h_hbm, idx_v,
             (r0, r1, r2, r3), acc_v,
             (s0, s1, s2, s3))


@jax.jit
def _pool(x_flat, emb_table):
    mesh = plsc.VectorSubcoreMesh(core_axis_name="c", subcore_axis_name="s")
    return pl.kernel(
        _pool_body,
        mesh=mesh,
        compiler_params=pltpu.CompilerParams(use_tc_tiling_on_sc=False),
        out_type=jax.ShapeDtypeStruct((B, HIDDEN), jnp.float32),
        scratch_types=(
            [pltpu.VMEM((BPW * LPAD,), jnp.int32)]
            + [pltpu.VMEM((CBL, HIDDEN), jnp.float32) for _ in range(NBUF)]
            + [pltpu.VMEM((BPW, HIDDEN), jnp.float32)]
            + [pltpu.SemaphoreType.DMA for _ in range(NBUF)]
        ),
    )(x_flat, emb_table)


def _mm_body(h_ref, w_ref, b_ref, o_ref):
    o_ref[...] = (
        lax.dot_general(
            h_ref[...], w_ref[...],
            dimension_numbers=(((1,), (1,)), ((), ())),
            preferred_element_type=jnp.float32,
        )
        + b_ref[...]
    )


@jax.jit
def _linear(h, W, b2d):
    bm = 512
    return pl.pallas_call(
        _mm_body,
        out_shape=jax.ShapeDtypeStruct((B, OUT), jnp.float32),
        grid=(B // bm,),
        in_specs=[
            pl.BlockSpec((bm, HIDDEN), lambda i: (i, 0)),
            pl.BlockSpec((OUT, HIDDEN), lambda i: (0, 0)),
            pl.BlockSpec((1, OUT), lambda i: (0, 0)),
        ],
        out_specs=pl.BlockSpec((bm, OUT), lambda i: (i, 0)),
    )(h, W, b2d)


def kernel(x, emb_table, W, b):
    x_flat = jnp.pad(x.astype(jnp.int32), ((0, 0), (0, LPAD - L))).reshape(-1)
    h = _pool(x_flat, emb_table)
    return _linear(h, W, b.reshape(1, OUT))
